# Initial kernel scaffold; baseline (speedup 1.0000x reference)
import jax
import jax.numpy as jnp
from jax.experimental import pallas as pl


def kernel(v, e, vidx, eidx, v_reg_weight, e_reg_weight, v_reg_sum, e_reg_sum, v_weight, W_v2e, W_e2v, b_v, b_e):
    raise NotImplementedError("write your pallas kernel here")


# speedup vs baseline: 3.8628x; 3.8628x over previous
import functools

import jax
import jax.numpy as jnp
from jax import lax
from jax.experimental import pallas as pl
from jax.experimental.pallas import tpu as pltpu
from jax.experimental.pallas import tpu_sc as plsc

_NC = 2
_NS = 16


def _matmul_scale_kernel(v_ref, w_ref, b_ref, s_ref, o_ref):
    acc = jnp.dot(v_ref[...], w_ref[...], preferred_element_type=jnp.float32)
    o_ref[...] = jnp.maximum(acc + b_ref[...], 0.0) * s_ref[...]


def _tc_vw(v, W, b2, v_weight, bm):
    n, d = v.shape
    return pl.pallas_call(
        _matmul_scale_kernel,
        grid=(n // bm,),
        in_specs=[
            pl.BlockSpec((bm, d), lambda i: (i, 0)),
            pl.BlockSpec((d, d), lambda i: (0, 0)),
            pl.BlockSpec((1, d), lambda i: (0, 0)),
            pl.BlockSpec((bm, 1), lambda i: (i, 0)),
        ],
        out_specs=pl.BlockSpec((bm, d), lambda i: (i, 0)),
        out_shape=jax.ShapeDtypeStruct((n, d), jnp.float32),
    )(v, W, b2, v_weight)


def _combine_matmul_kernel(e_ref, p_ref, rs_ref, w_ref, b_ref, eo_ref, ev_ref):
    es = (e_ref[...] + p_ref[...]) / rs_ref[...]
    eo_ref[...] = es
    acc = jnp.dot(es, w_ref[...], preferred_element_type=jnp.float32)
    ev_ref[...] = jnp.maximum(acc + b_ref[...], 0.0) / 3.0


def _tc_combine_matmul(e, p, e_rs, W, b2, bm):
    n, d = e.shape
    return pl.pallas_call(
        _combine_matmul_kernel,
        grid=(n // bm,),
        in_specs=[
            pl.BlockSpec((bm, d), lambda i: (i, 0)),
            pl.BlockSpec((bm, d), lambda i: (i, 0)),
            pl.BlockSpec((bm, 1), lambda i: (i, 0)),
            pl.BlockSpec((d, d), lambda i: (0, 0)),
            pl.BlockSpec((1, d), lambda i: (0, 0)),
        ],
        out_specs=[
            pl.BlockSpec((bm, d), lambda i: (i, 0)),
            pl.BlockSpec((bm, d), lambda i: (i, 0)),
        ],
        out_shape=[
            jax.ShapeDtypeStruct((n, d), jnp.float32),
            jax.ShapeDtypeStruct((n, d), jnp.float32),
        ],
    )(e, p, e_rs, W, b2)


def _vout_kernel(v_ref, vw_ref, q_ref, rs_ref, o_ref):
    o_ref[...] = (v_ref[...] * vw_ref[...] * 4.0 + q_ref[...]) / rs_ref[...]


def _tc_vout(v, v_weight, q, v_rs, bm):
    n, d = v.shape
    return pl.pallas_call(
        _vout_kernel,
        grid=(n // bm,),
        in_specs=[
            pl.BlockSpec((bm, d), lambda i: (i, 0)),
            pl.BlockSpec((bm, 1), lambda i: (i, 0)),
            pl.BlockSpec((bm, d), lambda i: (i, 0)),
            pl.BlockSpec((bm, 1), lambda i: (i, 0)),
        ],
        out_specs=pl.BlockSpec((bm, d), lambda i: (i, 0)),
        out_shape=jax.ShapeDtypeStruct((n, d), jnp.float32),
    )(v, v_weight, q, v_rs)


def _sc_phase(table, gidx4, sidx4, w4, zeros, half):
    ns, nch, cb, k = gidx4.shape
    d = table.shape[1]
    tr = half // _NS
    groups = k // 16

    @functools.partial(
        pl.kernel,
        out_type=jax.ShapeDtypeStruct((_NC * half, d), jnp.float32),
        mesh=plsc.VectorSubcoreMesh(core_axis_name="c", subcore_axis_name="s"),
        scratch_types=[
            pltpu.VMEM((cb, k), jnp.int32),
            pltpu.VMEM((cb, k), jnp.int32),
            pltpu.VMEM((cb, k), jnp.float32),
            pltpu.VMEM((1, k), jnp.int32),
            pltpu.VMEM((k, d), jnp.float32),
            pltpu.VMEM((k, d), jnp.float32),
            pltpu.VMEM_SHARED((half + _NS, d), jnp.float32),
            pltpu.SemaphoreType.DMA,
            pltpu.SemaphoreType.DMA,
        ],
    )
    def run(table_h, gidx_h, sidx_h, w_h, zeros_h, out_h,
            gidx_c, sidx_c, w_c, loc_v, rows0, rows1, acc, sem0, sem1):
        c = lax.axis_index("c")
        s = lax.axis_index("s")
        lo = c * half
        dump = half + s
        pltpu.sync_copy(zeros_h.at[pl.ds(s * tr, tr)], acc.at[pl.ds(s * tr, tr)])
        plsc.subcore_barrier()

        rows = (rows0, rows1)
        sems = (sem0, sem1)

        def scale_scatter(bi, buf):
            def group(g, carry):
                base = g * 16
                sl16 = pl.ds(base, 16)
                sv = sidx_c[bi, sl16] - lo
                ok = (sv >= 0) & (sv < half)
                loc_v[0, sl16] = jnp.where(ok, sv, dump)
                w16 = w_c[bi, sl16]
                for r in range(16):
                    wt = w16[r]
                    for j in range(d // 16):
                        sl = pl.ds(j * 16, 16)
                        buf[base + r, sl] = buf[base + r, sl] * wt
                return carry

            lax.fori_loop(0, groups, group, 0)
            pltpu.sync_copy(buf, acc.at[loc_v.at[0]], add=True)

        def chunk(ch, carry):
            pltpu.sync_copy(gidx_h.at[s, ch], gidx_c)
            pltpu.sync_copy(sidx_h.at[s, ch], sidx_c)
            pltpu.sync_copy(w_h.at[s, ch], w_c)
            copies = [None] * cb
            copies[0] = pltpu.async_copy(
                table_h.at[gidx_c.at[0]], rows[0], sems[0])
            for bi in range(cb):
                if bi + 1 < cb:
                    copies[bi + 1] = pltpu.async_copy(
                        table_h.at[gidx_c.at[bi + 1]],
                        rows[(bi + 1) % 2], sems[(bi + 1) % 2])
                copies[bi].wait()
                scale_scatter(bi, rows[bi % 2])
            return carry

        lax.fori_loop(0, nch, chunk, 0)
        plsc.subcore_barrier()
        pltpu.sync_copy(acc.at[pl.ds(s * tr, tr)],
                        out_h.at[pl.ds(lo + s * tr, tr)])

    return run(table, gidx4, sidx4, w4, zeros)


def _pick_batch(per_t):
    for k in range(128, 15, -16):
        if per_t % k == 0:
            return k
    return 16


def _pick_cb(nb):
    for cb in (5, 4, 2):
        if nb % cb == 0:
            return cb
    return 1


def kernel(v, e, vidx, eidx, v_reg_weight, e_reg_weight, v_reg_sum, e_reg_sum,
           v_weight, W_v2e, W_e2v, b_v, b_e):
    n_v, d = v.shape
    n_e = e.shape[0]
    nnz = vidx.shape[0]

    per_t = nnz // _NS
    k = _pick_batch(per_t)
    nb = per_t // k
    cb = _pick_cb(nb)
    nch = nb // cb
    align = 2 * 8 * _NS
    half_e = ((n_e + align - 1) // align) * (align // 2)
    half_v = ((n_v + align - 1) // align) * (align // 2)

    vidx4 = vidx.astype(jnp.int32).reshape(_NS, nch, cb, k)
    eidx4 = eidx.astype(jnp.int32).reshape(_NS, nch, cb, k)
    vrw4 = v_reg_weight.reshape(_NS, nch, cb, k)
    erw4 = e_reg_weight.reshape(_NS, nch, cb, k)
    zeros_e = jnp.zeros((half_e, d), jnp.float32)
    zeros_v = jnp.zeros((half_v, d), jnp.float32)
    b_v2 = b_v.reshape(1, d)
    b_e2 = b_e.reshape(1, d)

    vw = _tc_vw(v, W_v2e, b_v2, v_weight, bm=400)
    p_e = _sc_phase(vw, vidx4, eidx4, vrw4, zeros_e, half_e)
    e_out, ev3 = _tc_combine_matmul(e, p_e, e_reg_sum, W_e2v, b_e2, bm=1000)
    q_v = _sc_phase(ev3, eidx4, vidx4, erw4, zeros_v, half_v)
    v_out = _tc_vout(v, v_weight, q_v, v_reg_sum, bm=400)
    return (v_out, e_out)
